# Initial kernel scaffold; baseline (speedup 1.0000x reference)
import functools

import jax
import jax.numpy as jnp
from jax import lax
from jax.experimental import pallas as pl

N = 10000
E = 320000
D = 128
H = 128
B = 4096


def _gcn(x, src, dst, W, b):
    n = x.shape[0]
    loop = jnp.arange(n, dtype=src.dtype)
    s = jnp.concatenate([src, loop])
    d = jnp.concatenate([dst, loop])
    deg = jnp.zeros((n,), jnp.float32).at[d].add(1.0)
    dinv = jnp.where(deg > 0, lax.rsqrt(jnp.maximum(deg, 1e-12)), 0.0)
    xw = x @ W
    msg = xw[s] * (dinv[s] * dinv[d])[:, None]
    return jnp.zeros_like(xw).at[d].add(msg) + b


def _sage(x, src, dst, Wl, bl, Wr):
    n = x.shape[0]
    cnt = jnp.zeros((n,), jnp.float32).at[dst].add(1.0)
    ssum = jnp.zeros((n, x.shape[1]), jnp.float32).at[dst].add(x[src])
    mean = ssum / jnp.maximum(cnt, 1.0)[:, None]
    return mean @ Wl + bl + x @ Wr


def _gin(x, src, dst, W1, b1, W2, b2):
    agg = jnp.zeros_like(x).at[dst].add(x[src])
    h = x + agg
    h = jax.nn.relu(h @ W1 + b1)
    return h @ W2 + b2


def _gat(x, src, dst, W, a_s, a_d, b):
    n = x.shape[0]
    loop = jnp.arange(n, dtype=src.dtype)
    s = jnp.concatenate([src, loop])
    d = jnp.concatenate([dst, loop])
    xw = x @ W
    al_s = xw @ a_s
    al_d = xw @ a_d
    e = jax.nn.leaky_relu(al_s[s] + al_d[d], 0.2)
    m = jax.ops.segment_max(e, d, num_segments=n)
    ee = jnp.exp(e - m[d])
    den = jnp.zeros((n,), jnp.float32).at[d].add(ee)
    alpha = ee / jnp.maximum(den[d], 1e-16)
    out = jnp.zeros_like(xw).at[d].add(alpha[:, None] * xw[s])
    return out + b


def _layer(x, src, dst, p):
    x1 = _gcn(x, src, dst, p['gcn_W'], p['gcn_b'])
    x2 = _sage(x, src, dst, p['sage_Wl'], p['sage_bl'], p['sage_Wr'])
    x3 = _gin(x, src, dst, p['gin_W1'], p['gin_b1'], p['gin_W2'], p['gin_b2'])
    x4 = _gat(x, src, dst, p['gat_W'], p['gat_as'], p['gat_ad'], p['gat_b'])
    return jax.nn.relu(x1 + x2 + x3 + x4)


def _decode_tc(hu, hv, nu, nv, cu, cv, neoW, neob, W1u, W1v, w1n, b1, W2p, b2p):
    RB = 512

    def body(hu_r, hv_r, nu_r, nv_r, cu_r, cv_r, neoW_r, neob_r, W1u_r,
             W1v_r, w1n_r, b1_r, W2_r, b2_r, out_r):
        cu_b = cu_r[...][0]
        cv_b = cv_r[...][0]
        mu = jnp.where((cu_b > 0)[:, None],
                       nu_r[...] / jnp.maximum(cu_b, 1.0)[:, None], 0.0)
        mv = jnp.where((cv_b > 0)[:, None],
                       nv_r[...] / jnp.maximum(cv_b, 1.0)[:, None], 0.0)
        neo = jax.nn.sigmoid(
            jnp.dot(mu * mv, neoW_r[...], preferred_element_type=jnp.float32)
            + neob_r[...][0])
        h = (jnp.dot(hu_r[...], W1u_r[...], preferred_element_type=jnp.float32)
             + jnp.dot(hv_r[...], W1v_r[...], preferred_element_type=jnp.float32)
             + neo[:, :1] * w1n_r[...] + b1_r[...])
        h = jax.nn.relu(h)
        out_r[...] = (jnp.dot(h, W2_r[...], preferred_element_type=jnp.float32)
                      + b2_r[...])

    grid = (B // RB,)
    rspec = lambda: pl.BlockSpec((RB, 128), lambda i: (i, 0))
    cspec = lambda: pl.BlockSpec((1, RB), lambda i: (0, i))
    wspec = lambda shape: pl.BlockSpec(shape, lambda i: tuple(0 for _ in shape))
    out = pl.pallas_call(
        body,
        grid=grid,
        in_specs=[rspec(), rspec(), rspec(), rspec(), cspec(), cspec(),
                  wspec((128, 128)), wspec((1, 128)), wspec((128, 128)),
                  wspec((128, 128)), wspec((1, 128)), wspec((1, 128)),
                  wspec((128, 8)), wspec((1, 8))],
        out_specs=pl.BlockSpec((RB, 8), lambda i: (i, 0)),
        out_shape=jax.ShapeDtypeStruct((B, 8), jnp.float32),
    )(hu, hv, nu, nv, cu, cv, neoW, neob, W1u, W1v, w1n, b1, W2p, b2p)
    return out[:, :2]


def kernel(x, edge_index, edge_label_index, params):
    src, dst = edge_index[0], edge_index[1]
    z = x
    for p in params['layers']:
        z = _layer(z, src, dst, p)
    n = z.shape[0]
    adj_mask = jnp.zeros((n, n), dtype=bool).at[src, dst].set(True)
    src_l, dst_l = edge_label_index[0], edge_label_index[1]
    hu = z[src_l]
    hv = z[dst_l]
    rows_u = adj_mask[src_l]
    rows_v = adj_mask[dst_l]
    cu = rows_u.sum(axis=1).astype(jnp.float32)[None, :]
    cv = rows_v.sum(axis=1).astype(jnp.float32)[None, :]
    nu = rows_u.astype(jnp.float32) @ z
    nv = rows_v.astype(jnp.float32) @ z

    neoW = jnp.pad(params['neo_W'], ((0, 0), (0, 127)))
    neob = jnp.pad(params['neo_b'], (0, 127))[None, :]
    W1u = params['dec_W1'][:128]
    W1v = params['dec_W1'][128:256]
    w1n = params['dec_W1'][256:257]
    b1 = params['dec_b1'][None, :]
    W2p = jnp.pad(params['dec_W2'], ((0, 0), (0, 6)))
    b2p = jnp.pad(params['dec_b2'], (0, 6))[None, :]

    return _decode_tc(hu, hv, nu, nv, cu, cv, neoW, neob, W1u, W1v, w1n,
                      b1, W2p, b2p)


# speedup vs baseline: 1.0003x; 1.0003x over previous
import functools

import jax
import jax.numpy as jnp
from jax import lax
from jax.experimental import pallas as pl

N = 10000
E = 320000
D = 128
H = 128
B = 4096


def _gcn(x, src, dst, W, b):
    n = x.shape[0]
    loop = jnp.arange(n, dtype=src.dtype)
    s = jnp.concatenate([src, loop])
    d = jnp.concatenate([dst, loop])
    deg = jnp.zeros((n,), jnp.float32).at[d].add(1.0)
    dinv = jnp.where(deg > 0, lax.rsqrt(jnp.maximum(deg, 1e-12)), 0.0)
    xw = x @ W
    msg = xw[s] * (dinv[s] * dinv[d])[:, None]
    return jnp.zeros_like(xw).at[d].add(msg) + b


def _sage(x, src, dst, Wl, bl, Wr):
    n = x.shape[0]
    cnt = jnp.zeros((n,), jnp.float32).at[dst].add(1.0)
    ssum = jnp.zeros((n, x.shape[1]), jnp.float32).at[dst].add(x[src])
    mean = ssum / jnp.maximum(cnt, 1.0)[:, None]
    return mean @ Wl + bl + x @ Wr


def _gin(x, src, dst, W1, b1, W2, b2):
    agg = jnp.zeros_like(x).at[dst].add(x[src])
    h = x + agg
    h = jax.nn.relu(h @ W1 + b1)
    return h @ W2 + b2


def _gat(x, src, dst, W, a_s, a_d, b):
    n = x.shape[0]
    loop = jnp.arange(n, dtype=src.dtype)
    s = jnp.concatenate([src, loop])
    d = jnp.concatenate([dst, loop])
    xw = x @ W
    al_s = xw @ a_s
    al_d = xw @ a_d
    e = jax.nn.leaky_relu(al_s[s] + al_d[d], 0.2)
    m = jax.ops.segment_max(e, d, num_segments=n)
    ee = jnp.exp(e - m[d])
    den = jnp.zeros((n,), jnp.float32).at[d].add(ee)
    alpha = ee / jnp.maximum(den[d], 1e-16)
    out = jnp.zeros_like(xw).at[d].add(alpha[:, None] * xw[s])
    return out + b


def _layer(x, src, dst, p):
    x1 = _gcn(x, src, dst, p['gcn_W'], p['gcn_b'])
    x2 = _sage(x, src, dst, p['sage_Wl'], p['sage_bl'], p['sage_Wr'])
    x3 = _gin(x, src, dst, p['gin_W1'], p['gin_b1'], p['gin_W2'], p['gin_b2'])
    x4 = _gat(x, src, dst, p['gat_W'], p['gat_as'], p['gat_ad'], p['gat_b'])
    return jax.nn.relu(x1 + x2 + x3 + x4)


def _decode_tc(hu, hv, nu, nv, cu, cv, neoW, neob, W1u, W1v, w1n, b1, W2p, b2p):
    RB = 512

    def body(hu_r, hv_r, nu_r, nv_r, cu_r, cv_r, neoW_r, neob_r, W1u_r,
             W1v_r, w1n_r, b1_r, W2_r, b2_r, out_r):
        cu_b = cu_r[...]
        cv_b = cv_r[...]
        mu = nu_r[...] / jnp.maximum(cu_b, 1.0)
        mv = nv_r[...] / jnp.maximum(cv_b, 1.0)
        neo = jax.nn.sigmoid(
            jnp.dot(mu * mv, neoW_r[...], preferred_element_type=jnp.float32)
            + neob_r[...][0])
        h = (jnp.dot(hu_r[...], W1u_r[...], preferred_element_type=jnp.float32)
             + jnp.dot(hv_r[...], W1v_r[...], preferred_element_type=jnp.float32)
             + neo[:, :1] * w1n_r[...] + b1_r[...])
        h = jax.nn.relu(h)
        out_r[...] = (jnp.dot(h, W2_r[...], preferred_element_type=jnp.float32)
                      + b2_r[...])

    grid = (B // RB,)
    rspec = lambda: pl.BlockSpec((RB, 128), lambda i: (i, 0))
    cspec = lambda: pl.BlockSpec((RB, 1), lambda i: (i, 0))
    wspec = lambda shape: pl.BlockSpec(shape, lambda i: tuple(0 for _ in shape))
    out = pl.pallas_call(
        body,
        grid=grid,
        in_specs=[rspec(), rspec(), rspec(), rspec(), cspec(), cspec(),
                  wspec((128, 128)), wspec((1, 128)), wspec((128, 128)),
                  wspec((128, 128)), wspec((1, 128)), wspec((1, 128)),
                  wspec((128, 8)), wspec((1, 8))],
        out_specs=pl.BlockSpec((RB, 8), lambda i: (i, 0)),
        out_shape=jax.ShapeDtypeStruct((B, 8), jnp.float32),
    )(hu, hv, nu, nv, cu, cv, neoW, neob, W1u, W1v, w1n, b1, W2p, b2p)
    return out[:, :2]


def kernel(x, edge_index, edge_label_index, params):
    src, dst = edge_index[0], edge_index[1]
    z = x
    for p in params['layers']:
        z = _layer(z, src, dst, p)
    n = z.shape[0]
    adj_mask = jnp.zeros((n, n), dtype=bool).at[src, dst].set(True)
    src_l, dst_l = edge_label_index[0], edge_label_index[1]
    hu = z[src_l]
    hv = z[dst_l]
    rows_u = adj_mask[src_l]
    rows_v = adj_mask[dst_l]
    cu = rows_u.sum(axis=1).astype(jnp.float32)[:, None]
    cv = rows_v.sum(axis=1).astype(jnp.float32)[:, None]
    nu = rows_u.astype(jnp.float32) @ z
    nv = rows_v.astype(jnp.float32) @ z

    neoW = jnp.pad(params['neo_W'], ((0, 0), (0, 127)))
    neob = jnp.pad(params['neo_b'], (0, 127))[None, :]
    W1u = params['dec_W1'][:128]
    W1v = params['dec_W1'][128:256]
    w1n = params['dec_W1'][256:257]
    b1 = params['dec_b1'][None, :]
    W2p = jnp.pad(params['dec_W2'], ((0, 0), (0, 6)))
    b2p = jnp.pad(params['dec_b2'], (0, 6))[None, :]

    return _decode_tc(hu, hv, nu, nv, cu, cv, neoW, neob, W1u, W1v, w1n,
                      b1, W2p, b2p)


# speedup vs baseline: 6.6837x; 6.6814x over previous
import functools

import jax
import jax.numpy as jnp
from jax import lax
from jax.experimental import pallas as pl
from jax.experimental.pallas import tpu as pltpu
from jax.experimental.pallas import tpu_sc as plsc

N = 10000
E = 320000
B = 4096
NP = 10240
NPT = 320
NT = 32
EPAD = E + 2048
SH = 14
MSK = (1 << SH) - 1
F32 = jnp.float32
I32 = jnp.int32

_MESH = plsc.VectorSubcoreMesh(core_axis_name="c", subcore_axis_name="s",
                               num_cores=2)


def _wid():
    return lax.axis_index("s") * 2 + lax.axis_index("c")


def _leaky(t):
    return jnp.maximum(t, 0.2 * t)


def _sload(ref, i):
    return ref[pl.ds(i, 16)][0]


def _round_bf16(v):
    u = plsc.bitcast(v, jnp.int32)
    lsb = (u >> 16) & 1
    u2 = (u + 32767 + lsb) & (-65536)
    return plsc.bitcast(u2, F32)


def _sstore(ref, i, val):
    idx = lax.full((16,), 0, I32) + i
    v = lax.full((16,), 0, F32) + val
    msk = lax.iota(I32, 16) == 0
    plsc.store_scatter(ref, [idx], v, mask=msk)


def _sc_scalar(als, ald, ssrc, sdst, tb):
    CS = 512

    def body(als_h, ald_h, ssrc_h, sdst_h, tb_h, m_h, den_h, ees_h,
             als_v, ald_v, tb_v, sbuf, dbuf, ebuf, eebuf, mst, denst, eest):
        w = _wid()
        n0 = w * NPT
        pltpu.sync_copy(als_h, als_v)
        pltpu.sync_copy(ald_h.at[pl.ds(n0, NPT)], ald_v.at[pl.ds(0, NPT)])
        pltpu.sync_copy(tb_h, tb_v.at[pl.ds(0, 48)])
        for j in range(NPT // 16):
            a = als_v[pl.ds(n0 + j * 16, 16)]
            b = ald_v[pl.ds(j * 16, 16)]
            mst[pl.ds(j * 16, 16)] = _leaky(a + b)
            denst[pl.ds(j * 16, 16)] = jnp.zeros((16,), F32)
        e0 = _sload(tb_v, w)
        e1 = _sload(tb_v, w + 1)
        a0 = (e0 // 16) * 16
        nch = (e1 - a0 + CS - 1) // CS

        def _chunk_logits(base):
            pltpu.sync_copy(ssrc_h.at[pl.ds(base, CS)], sbuf.at[pl.ds(0, CS)])
            pltpu.sync_copy(sdst_h.at[pl.ds(base, CS)], dbuf.at[pl.ds(0, CS)])
            for j in range(CS // 16):
                sv = sbuf[pl.ds(j * 16, 16)]
                dv = dbuf[pl.ds(j * 16, 16)]
                dl = jnp.clip(dv - n0, 0, NPT - 1)
                av = plsc.load_gather(als_v, [sv])
                bv = plsc.load_gather(ald_v, [dl])
                ebuf[pl.ds(j * 16, 16)] = _leaky(av + bv)

        def chA(k, carry):
            base = a0 + k * CS
            _chunk_logits(base)
            lo = jnp.maximum(e0 - base, 0)
            hi = jnp.minimum(e1 - base, CS)

            def eA(e, c):
                cur, mcur = c
                d = _sload(dbuf, e)
                ev = _sload(ebuf, e)

                def chg(_):
                    _sstore(mst, cur - n0, mcur)
                    return d, _sload(mst, d - n0)

                cur, mcur = lax.cond(d != cur, chg,
                                     lambda _: (cur, mcur), 0)
                return cur, jnp.maximum(mcur, ev)

            return lax.fori_loop(lo, hi, eA, carry)

        cur, mcur = lax.fori_loop(0, nch, chA, (n0, _sload(mst, 0)))
        _sstore(mst, cur - n0, mcur)

        for j in range(NPT // 16):
            a = als_v[pl.ds(n0 + j * 16, 16)]
            b = ald_v[pl.ds(j * 16, 16)]
            es = _leaky(a + b)
            eest[pl.ds(j * 16, 16)] = jnp.exp(es - mst[pl.ds(j * 16, 16)])

        def chB(k, carry):
            base = a0 + k * CS
            _chunk_logits(base)
            for j in range(CS // 16):
                dv = dbuf[pl.ds(j * 16, 16)]
                dl = jnp.clip(dv - n0, 0, NPT - 1)
                mv = plsc.load_gather(mst, [dl])
                eebuf[pl.ds(j * 16, 16)] = jnp.exp(
                    ebuf[pl.ds(j * 16, 16)] - mv)
            lo = jnp.maximum(e0 - base, 0)
            hi = jnp.minimum(e1 - base, CS)

            def eB(e, c):
                cur, scur = c
                d = _sload(dbuf, e)
                ev = _sload(eebuf, e)

                def chg(_):
                    _sstore(denst, cur - n0, scur)
                    return d, jnp.float32(0.0)

                cur, scur = lax.cond(d != cur, chg,
                                     lambda _: (cur, scur), 0)
                return cur, scur + ev

            return lax.fori_loop(lo, hi, eB, carry)

        cur, scur = lax.fori_loop(0, nch, chB, (n0, jnp.float32(0.0)))
        _sstore(denst, cur - n0, scur)
        pltpu.sync_copy(mst.at[pl.ds(0, NPT)], m_h.at[pl.ds(n0, NPT)])
        pltpu.sync_copy(denst.at[pl.ds(0, NPT)], den_h.at[pl.ds(n0, NPT)])
        pltpu.sync_copy(eest, ees_h.at[pl.ds(n0, NPT)])

    f = pl.kernel(
        body,
        out_type=[jax.ShapeDtypeStruct((NP,), F32)] * 3,
        mesh=_MESH,
        compiler_params=pltpu.CompilerParams(needs_layout_passes=False),
        scratch_types=[
            pltpu.VMEM((NP,), F32), pltpu.VMEM((NPT + 16,), F32),
            pltpu.VMEM((64,), I32),
            pltpu.VMEM((CS + 16,), I32), pltpu.VMEM((CS + 16,), I32),
            pltpu.VMEM((CS + 16,), F32), pltpu.VMEM((CS + 16,), F32),
            pltpu.VMEM((NPT + 16,), F32), pltpu.VMEM((NPT + 16,), F32),
            pltpu.VMEM((NPT,), F32),
        ],
    )
    return f(als, ald, ssrc, sdst, tb)


def _sc_accum(zcat, ssrc, sdst, tbq, dinv, als, ald, m):
    CA = 64
    NQ = 40

    def body(zc_h, ssrc_h, sdst_h, tbq_h, dinv_h, als_h, ald_h, m_h,
             s0_h, s1_h, s2_h,
             dinv_v, als_v, ald_v, m_v, tb_v, sbuf, dbuf, w1buf, w2buf,
             rows, st0, st1, st2, sem):
        outs = (s0_h, s1_h, s2_h)
        w = _wid()
        n0 = w * NPT
        pltpu.sync_copy(dinv_h, dinv_v)
        pltpu.sync_copy(als_h, als_v)
        pltpu.sync_copy(ald_h.at[pl.ds(n0, NPT)], ald_v.at[pl.ds(0, NPT)])
        pltpu.sync_copy(m_h.at[pl.ds(n0, NPT)], m_v.at[pl.ds(0, NPT)])
        pltpu.sync_copy(tbq_h, tb_v.at[pl.ds(0, 272)])
        zero = jnp.zeros((16,), F32)
        sts = (st0, st1, st2)

        def zr(i, _):
            for j in range(8):
                st0[i, pl.ds(j * 16, 16)] = zero
                st1[i, pl.ds(j * 16, 16)] = zero
                st2[i, pl.ds(j * 16, 16)] = zero
            return 0

        def qpass(q, _):
            n0q = n0 + q * NQ
            lax.fori_loop(0, NQ, zr, 0)
            e0 = _sload(tb_v, w * 8 + q)
            e1 = _sload(tb_v, w * 8 + q + 1)
            a0 = (e0 // 16) * 16
            nch = (e1 - a0 + CA - 1) // CA

            def ch(k, carry):
                base = a0 + k * CA
                pltpu.sync_copy(ssrc_h.at[pl.ds(base, CA)],
                                sbuf.at[pl.ds(0, CA)])
                pltpu.sync_copy(sdst_h.at[pl.ds(base, CA)],
                                dbuf.at[pl.ds(0, CA)])
                pltpu.async_copy(zc_h.at[sbuf.at[pl.ds(0, CA)]], rows,
                                 sem).wait()
                for j in range(CA // 16):
                    sv = sbuf[pl.ds(j * 16, 16)]
                    dv = dbuf[pl.ds(j * 16, 16)]
                    dl = jnp.clip(dv - n0, 0, NPT - 1)
                    w1buf[pl.ds(j * 16, 16)] = plsc.load_gather(
                        dinv_v, [sv])
                    av = plsc.load_gather(als_v, [sv])
                    bv = plsc.load_gather(ald_v, [dl])
                    mv = plsc.load_gather(m_v, [dl])
                    w2buf[pl.ds(j * 16, 16)] = jnp.exp(
                        _leaky(av + bv) - mv)
                lo = jnp.maximum(e0 - base, 0)
                hi = jnp.minimum(e1 - base, CA)

                def ee(e, c):
                    cur = c[0]
                    acc = c[1]
                    d = _sload(dbuf, e)

                    def chg(_):
                        r = cur - n0q
                        for a in range(3):
                            for j in range(8):
                                sts[a][r, pl.ds(j * 16, 16)] = (
                                    acc[a * 8 + j])
                        return d, (zero,) * 24

                    cur2, acc = lax.cond(d != cur, chg,
                                         lambda _: (cur, acc), 0)
                    w1s = _sload(w1buf, e)
                    w2s = _sload(w2buf, e)
                    na = []
                    for j in range(8):
                        rj = rows[e, pl.ds(j * 16, 16)]
                        na.append(acc[j] + rj)
                    for j in range(8):
                        rj = rows[e, pl.ds(128 + j * 16, 16)]
                        na.append(acc[8 + j] + w1s * rj)
                    for j in range(8):
                        rj = rows[e, pl.ds(256 + j * 16, 16)]
                        na.append(acc[16 + j] + w2s * rj)
                    return cur2, tuple(na)

                return lax.fori_loop(lo, hi, ee, carry)

            cur, acc = lax.fori_loop(0, nch, ch, (n0q, (zero,) * 24))
            r = cur - n0q
            for a in range(3):
                for j in range(8):
                    sts[a][r, pl.ds(j * 16, 16)] = acc[a * 8 + j]
            for a in range(3):
                pltpu.sync_copy(sts[a], outs[a].at[pl.ds(n0q, NQ)])
            return 0

        lax.fori_loop(0, 8, qpass, 0)

    f = pl.kernel(
        body,
        out_type=[jax.ShapeDtypeStruct((NP, 128), F32)] * 3,
        mesh=_MESH,
        compiler_params=pltpu.CompilerParams(needs_layout_passes=False),
        scratch_types=[
            pltpu.VMEM((NP,), F32), pltpu.VMEM((NP,), F32),
            pltpu.VMEM((NPT + 16,), F32), pltpu.VMEM((NPT + 16,), F32),
            pltpu.VMEM((272,), I32),
            pltpu.VMEM((CA + 16,), I32), pltpu.VMEM((CA + 16,), I32),
            pltpu.VMEM((CA + 16,), F32), pltpu.VMEM((CA + 16,), F32),
            pltpu.VMEM((CA, 384), F32),
            pltpu.VMEM((NQ, 128), F32), pltpu.VMEM((NQ, 128), F32),
            pltpu.VMEM((NQ, 128), F32),
            pltpu.SemaphoreType.DMA,
        ],
    )
    return f(zcat, ssrc, sdst, tbq, dinv, als, ald, m)


def _sc_dedupe(z, keys, keyprev, tbq):
    CD = 128
    NQ = 80

    def body(z_h, keys_h, kprev_h, tbq_h, nb_h, cnt_h,
             tb_v, kbuf, kpbuf, dbuf, wbuf, rows, nbst, cntst, sem):
        w = _wid()
        n0 = w * NPT
        pltpu.sync_copy(tbq_h, tb_v.at[pl.ds(0, 144)])
        zero = jnp.zeros((16,), F32)

        def zr(i, _):
            for j in range(8):
                nbst[i, pl.ds(j * 16, 16)] = zero
            return 0

        def qpass(q, _):
            n0q = n0 + q * NQ
            lax.fori_loop(0, NQ, zr, 0)
            for i in range(NQ // 16):
                cntst[pl.ds(i * 16, 16)] = zero
            e0 = _sload(tb_v, w * 4 + q)
            e1 = _sload(tb_v, w * 4 + q + 1)
            a0 = (e0 // 16) * 16
            nch = (e1 - a0 + CD - 1) // CD

            def ch(k, carry):
                base = a0 + k * CD
                pltpu.sync_copy(keys_h.at[pl.ds(base, CD)],
                                kbuf.at[pl.ds(0, CD)])
                pltpu.sync_copy(kprev_h.at[pl.ds(base, CD)],
                                kpbuf.at[pl.ds(0, CD)])
                for j in range(CD // 16):
                    kv = kbuf[pl.ds(j * 16, 16)]
                    kp = kpbuf[pl.ds(j * 16, 16)]
                    dbuf[pl.ds(j * 16, 16)] = kv & MSK
                    wbuf[pl.ds(j * 16, 16)] = jnp.where(
                        kv != kp, jnp.full((16,), 1.0, F32),
                        jnp.zeros((16,), F32))
                pltpu.async_copy(z_h.at[dbuf.at[pl.ds(0, CD)]], rows,
                                 sem).wait()
                lo = jnp.maximum(e0 - base, 0)
                hi = jnp.minimum(e1 - base, CD)

                def ee(e, c):
                    cur = c[0]
                    acc = c[1]
                    ccur = c[2]
                    s = _sload(kbuf, e) >> SH

                    def chg(_):
                        r = cur - n0q
                        for j in range(8):
                            nbst[r, pl.ds(j * 16, 16)] = acc[j]
                        _sstore(cntst, r, ccur)
                        return s, (zero,) * 8, jnp.float32(0.0)

                    cur2, acc, ccur = lax.cond(s != cur, chg,
                                               lambda _: (cur, acc, ccur), 0)
                    ws = _sload(wbuf, e)
                    na = []
                    for j in range(8):
                        rv = _round_bf16(rows[e, pl.ds(j * 16, 16)])
                        na.append(acc[j] + ws * rv)
                    return cur2, tuple(na), ccur + ws

                return lax.fori_loop(lo, hi, ee, carry)

            cur, acc, ccur = lax.fori_loop(
                0, nch, ch, (n0q, (zero,) * 8, jnp.float32(0.0)))
            r = cur - n0q
            for j in range(8):
                nbst[r, pl.ds(j * 16, 16)] = acc[j]
            _sstore(cntst, r, ccur)
            pltpu.sync_copy(nbst, nb_h.at[pl.ds(n0q, NQ)])
            pltpu.sync_copy(cntst.at[pl.ds(0, NQ)], cnt_h.at[pl.ds(n0q, NQ)])
            return 0

        lax.fori_loop(0, 4, qpass, 0)

    f = pl.kernel(
        body,
        out_type=[jax.ShapeDtypeStruct((NP, 128), F32),
                  jax.ShapeDtypeStruct((NP,), F32)],
        mesh=_MESH,
        compiler_params=pltpu.CompilerParams(needs_layout_passes=False),
        scratch_types=[
            pltpu.VMEM((160,), I32),
            pltpu.VMEM((CD + 16,), I32), pltpu.VMEM((CD + 16,), I32),
            pltpu.VMEM((CD + 16,), I32), pltpu.VMEM((CD + 16,), F32),
            pltpu.VMEM((CD, 128), F32),
            pltpu.VMEM((NQ, 128), F32), pltpu.VMEM((NQ + 16,), F32),
            pltpu.SemaphoreType.DMA,
        ],
    )
    return f(z, keys, keyprev, tbq)


def _sc_qgather(z, nb, cntd, q):
    QW = (2 * B) // NT
    CQ = 32

    def body(z_h, nb_h, cnt_h, q_h, hq_h, nq_h, cq_h,
             cnt_v, qbuf, hrows, nrows, cbuf, sem):
        w = _wid()
        q0 = w * QW
        pltpu.sync_copy(cnt_h, cnt_v)
        for c in range(QW // CQ):
            base = q0 + c * CQ
            pltpu.sync_copy(q_h.at[pl.ds(base, CQ)], qbuf)
            pltpu.async_copy(z_h.at[qbuf], hrows, sem).wait()
            pltpu.async_copy(nb_h.at[qbuf], nrows, sem).wait()
            for j in range(CQ // 16):
                qv = qbuf[pl.ds(j * 16, 16)]
                cbuf[pl.ds(j * 16, 16)] = plsc.load_gather(cnt_v, [qv])
            pltpu.sync_copy(hrows, hq_h.at[pl.ds(base, CQ)])
            pltpu.sync_copy(nrows, nq_h.at[pl.ds(base, CQ)])
            pltpu.sync_copy(cbuf, cq_h.at[pl.ds(base, CQ)])

    f = pl.kernel(
        body,
        out_type=[jax.ShapeDtypeStruct((2 * B, 128), F32),
                  jax.ShapeDtypeStruct((2 * B, 128), F32),
                  jax.ShapeDtypeStruct((2 * B,), F32)],
        mesh=_MESH,
        compiler_params=pltpu.CompilerParams(needs_layout_passes=False),
        scratch_types=[
            pltpu.VMEM((NP,), F32),
            pltpu.VMEM((CQ,), I32),
            pltpu.VMEM((CQ, 128), F32), pltpu.VMEM((CQ, 128), F32),
            pltpu.VMEM((CQ,), F32),
            pltpu.SemaphoreType.DMA,
        ],
    )
    return f(z, nb, cntd, q)


def _tc_prep(xp, cnt2, wgcn0, wg0, asd0):
    R = 1024

    def body(x_r, cnt_r, wgcn_r, wg_r, asd_r, zc_r, aa_r, dinv_r, invc_r):
        c = cnt_r[...]
        dinv_r[...] = lax.rsqrt(c + 1.0)
        invc_r[...] = 1.0 / jnp.maximum(c, 1.0)
        xb = x_r[...]
        xw = jnp.dot(xb, wgcn_r[...], preferred_element_type=F32)
        xwg = jnp.dot(xb, wg_r[...], preferred_element_type=F32)
        zc_r[...] = jnp.concatenate([xb, xw, xwg], axis=1)
        aa_r[...] = jnp.dot(xwg, asd_r[...], preferred_element_type=F32)

    return pl.pallas_call(
        body,
        grid=(NP // R,),
        in_specs=[pl.BlockSpec((R, 128), lambda i: (i, 0)),
                  pl.BlockSpec((R, 1), lambda i: (i, 0)),
                  pl.BlockSpec((128, 128), lambda i: (0, 0)),
                  pl.BlockSpec((128, 128), lambda i: (0, 0)),
                  pl.BlockSpec((128, 8), lambda i: (0, 0))],
        out_specs=[pl.BlockSpec((R, 384), lambda i: (i, 0)),
                   pl.BlockSpec((R, 8), lambda i: (i, 0)),
                   pl.BlockSpec((R, 1), lambda i: (i, 0)),
                   pl.BlockSpec((R, 1), lambda i: (i, 0))],
        out_shape=[jax.ShapeDtypeStruct((NP, 384), F32),
                   jax.ShapeDtypeStruct((NP, 8), F32),
                   jax.ShapeDtypeStruct((NP, 1), F32),
                   jax.ShapeDtypeStruct((NP, 1), F32)],
    )(xp, cnt2, wgcn0, wg0, asd0)


def _tc_combine(z, zcat, Sh, dinv2, invc2, den2, ees2, lp,
                wgcn_next, wg_next, asd_next):
    R = 1024
    b1 = lp['gin_b1'][None, :]
    bsum = (lp['gcn_b'] + lp['sage_bl'] + lp['gin_b2'] + lp['gat_b'])[None, :]

    def body(z_r, zc_r, s0_r, s1_r, s2_r, dinv_r, invc_r, den_r, ees_r,
             wl_r, wr_r, wg1_r, wg2_r, b1_r, bsum_r,
             wgcnn_r, wgn_r, asd_r, zn_r, zcn_r, aa_r):
        zb = z_r[...]
        xw = zc_r[...][:, 128:256]
        xwg = zc_r[...][:, 256:384]
        s0 = s0_r[...]
        dinv = dinv_r[...]
        ees = ees_r[...]
        h1 = jax.nn.relu(jnp.dot(zb + s0, wg1_r[...],
                                 preferred_element_type=F32) + b1_r[...])
        g1 = dinv * s1_r[...] + (dinv * dinv) * xw
        g2 = s0 * invc_r[...]
        dent = jnp.maximum(den_r[...] + ees, 1e-16)
        g4 = (s2_r[...] + ees * xwg) / dent
        o = (g1
             + jnp.dot(g2, wl_r[...], preferred_element_type=F32)
             + jnp.dot(zb, wr_r[...], preferred_element_type=F32)
             + jnp.dot(h1, wg2_r[...], preferred_element_type=F32)
             + g4
             + bsum_r[...])
        zn = jax.nn.relu(o)
        zn_r[...] = zn
        xwn = jnp.dot(zn, wgcnn_r[...], preferred_element_type=F32)
        xwgn = jnp.dot(zn, wgn_r[...], preferred_element_type=F32)
        zcn_r[...] = jnp.concatenate([zn, xwn, xwgn], axis=1)
        aa_r[...] = jnp.dot(xwgn, asd_r[...], preferred_element_type=F32)

    rs = lambda: pl.BlockSpec((R, 128), lambda i: (i, 0))
    zs = lambda: pl.BlockSpec((R, 384), lambda i: (i, 0))
    cs = lambda: pl.BlockSpec((R, 1), lambda i: (i, 0))
    ws = lambda s: pl.BlockSpec(s, lambda i: (0, 0))
    return pl.pallas_call(
        body,
        grid=(NP // R,),
        in_specs=[rs(), zs(), rs(), rs(), rs(),
                  cs(), cs(), cs(), cs(),
                  ws((128, 128)), ws((128, 128)), ws((128, 128)),
                  ws((128, 128)), ws((1, 128)), ws((1, 128)),
                  ws((128, 128)), ws((128, 128)), ws((128, 8))],
        out_specs=[rs(), zs(), pl.BlockSpec((R, 8), lambda i: (i, 0))],
        out_shape=[jax.ShapeDtypeStruct((NP, 128), F32),
                   jax.ShapeDtypeStruct((NP, 384), F32),
                   jax.ShapeDtypeStruct((NP, 8), F32)],
    )(z, zcat, *Sh, dinv2, invc2, den2, ees2,
      lp['sage_Wl'], lp['sage_Wr'], lp['gin_W1'], lp['gin_W2'], b1, bsum,
      wgcn_next, wg_next, asd_next)


def _tc_decode(hu, hv, nu, nv, cu, cv, neoW, neob, W1u, W1v, w1n, b1, W2p,
               b2p):
    RB = 512

    def body(hu_r, hv_r, nu_r, nv_r, cu_r, cv_r, neoW_r, neob_r, W1u_r,
             W1v_r, w1n_r, b1_r, W2_r, b2_r, out_r):
        mu = nu_r[...] / jnp.maximum(cu_r[...], 1.0)
        mv = nv_r[...] / jnp.maximum(cv_r[...], 1.0)
        neo = jax.nn.sigmoid(
            jnp.dot(mu * mv, neoW_r[...], preferred_element_type=F32)
            + neob_r[...][0])
        h = (jnp.dot(hu_r[...], W1u_r[...], preferred_element_type=F32)
             + jnp.dot(hv_r[...], W1v_r[...], preferred_element_type=F32)
             + neo[:, :1] * w1n_r[...] + b1_r[...])
        h = jax.nn.relu(h)
        out_r[...] = (jnp.dot(h, W2_r[...], preferred_element_type=F32)
                      + b2_r[...])

    rspec = lambda: pl.BlockSpec((RB, 128), lambda i: (i, 0))
    cspec = lambda: pl.BlockSpec((RB, 1), lambda i: (i, 0))
    wspec = lambda s: pl.BlockSpec(s, lambda i: (0, 0))
    out = pl.pallas_call(
        body,
        grid=(B // RB,),
        in_specs=[rspec(), rspec(), rspec(), rspec(), cspec(), cspec(),
                  wspec((128, 128)), wspec((1, 128)), wspec((128, 128)),
                  wspec((128, 128)), wspec((1, 128)), wspec((1, 128)),
                  wspec((128, 8)), wspec((1, 8))],
        out_specs=pl.BlockSpec((RB, 8), lambda i: (i, 0)),
        out_shape=jax.ShapeDtypeStruct((B, 8), F32),
    )(hu, hv, nu, nv, cu, cv, neoW, neob, W1u, W1v, w1n, b1, W2p, b2p)
    return out[:, :2]


def kernel(x, edge_index, edge_label_index, params):
    src, dst = edge_index[0], edge_index[1]
    xp = jnp.pad(x, ((0, NP - N), (0, 0)))

    key1s = jnp.sort(dst * (1 << SH) + src)
    sdst1 = key1s >> SH
    ssrc1 = key1s & MSK
    key2s = jnp.sort(src * (1 << SH) + dst)
    nodes = jnp.arange(NP + 1, dtype=I32)
    starts1 = jnp.searchsorted(key1s, nodes << SH).astype(I32)
    starts2 = jnp.searchsorted(key2s, nodes << SH).astype(I32)
    cnt2 = (starts1[1:] - starts1[:-1]).astype(F32)[:, None]
    tb1 = jnp.pad(starts1[::NPT], (0, 15), mode='edge')
    tbq1 = jnp.pad(starts1[::40], (0, 15), mode='edge')
    tbq2 = jnp.pad(starts2[::80], (0, 15), mode='edge')
    ssrc1p = jnp.pad(ssrc1, (0, EPAD - E))
    sdst1p = jnp.pad(sdst1, (0, EPAD - E))
    key2sp = jnp.pad(key2s, (0, EPAD - E))
    keyprevp = jnp.pad(
        jnp.concatenate([jnp.full((1,), -1, I32), key2s[:-1]]),
        (0, EPAD - E))

    layers = params['layers']
    asd = []
    wgs = []
    wgcns = []
    for lp in layers:
        v = jnp.stack([lp['gat_as'], lp['gat_ad']], axis=1)
        asd.append(jnp.pad(v, ((0, 0), (0, 6))))
        wgs.append(lp['gat_W'])
        wgcns.append(lp['gcn_W'])
    asd.append(jnp.zeros((128, 8), F32))
    wgs.append(jnp.zeros((128, 128), F32))
    wgcns.append(jnp.zeros((128, 128), F32))

    zcat, alsald, dinv2, invc2 = _tc_prep(xp, cnt2, wgcns[0], wgs[0], asd[0])
    dinv1 = dinv2.reshape(NP)
    z = xp
    for l in range(3):
        als = alsald[:, 0]
        ald = alsald[:, 1]
        m, den, ees = _sc_scalar(als, ald, ssrc1p, sdst1p, tb1)
        Sh = _sc_accum(zcat, ssrc1p, sdst1p, tbq1, dinv1, als, ald, m)
        z, zcat, alsald = _tc_combine(
            z, zcat, Sh, dinv2, invc2, den[:, None], ees[:, None],
            layers[l], wgcns[l + 1], wgs[l + 1], asd[l + 1])

    NB, cntd = _sc_dedupe(z, key2sp, keyprevp, tbq2)
    q = jnp.concatenate([edge_label_index[0], edge_label_index[1]])
    hq, nq, cq = _sc_qgather(z, NB, cntd, q)

    neoW = jnp.pad(params['neo_W'], ((0, 0), (0, 127)))
    neob = jnp.pad(params['neo_b'], (0, 127))[None, :]
    W1u = params['dec_W1'][:128]
    W1v = params['dec_W1'][128:256]
    w1n = params['dec_W1'][256:257]
    b1 = params['dec_b1'][None, :]
    W2p = jnp.pad(params['dec_W2'], ((0, 0), (0, 6)))
    b2p = jnp.pad(params['dec_b2'], (0, 6))[None, :]

    return _tc_decode(hq[:B], hq[B:], nq[:B], nq[B:],
                      cq[:B, None], cq[B:, None],
                      neoW, neob, W1u, W1v, w1n, b1, W2p, b2p)
